# baseline (speedup 1.0000x reference)
import jax
import jax.numpy as jnp
from jax import lax
from jax.experimental import pallas as pl
from jax.experimental.pallas import tpu as pltpu
from jax.experimental.pallas import tpu_sc as plsc

_NC = 2
_NS = 16
_NW = _NC * _NS
_C = 128
_PAD = 8


def _edge_layout(E):
  e_per_tile = -(-E // _NW)
  n_chunks = -(-e_per_tile // _C)
  n_chunks = ((n_chunks + 7) // 8) * 8
  if n_chunks % 2:
    n_chunks += 1
  return n_chunks, _NW * n_chunks * _C


def _stripes(NP):
  RS = 8 * ((NP + 8 * _NS - 1) // (8 * _NS))
  RS_LAST = NP - RS * (_NS - 1)
  assert 0 < RS_LAST <= RS
  return RS, RS_LAST


def _make_agg(N, E, W):
  NP = N + _PAD
  n_chunks, EP = _edge_layout(E)
  C = _C
  RS, RS_LAST = _stripes(NP)

  mesh = plsc.VectorSubcoreMesh(core_axis_name="c", subcore_axis_name="s")

  def body(x_hbm, srcr_hbm, dst_hbm, z_hbm, out_hbm, acc,
           src_i, d0, d1, rows0, rows1, sem0, sem1, semd0, semd1):
    cid = lax.axis_index("c")
    sid = lax.axis_index("s")
    wid = sid * _NC + cid
    r0 = sid * RS
    ebase = wid * n_chunks * C

    def stripe_chunks(L, fn):
      off = 0
      while off < L:
        ln = min(C, L - off)
        fn(off, ln)
        off += ln

    def per_stripe(fn):
      @pl.when(sid != _NS - 1)
      def _():
        fn(RS)

      @pl.when(sid == _NS - 1)
      def _():
        fn(RS_LAST)

    pltpu.sync_copy(srcr_hbm.at[pl.ds(wid * n_chunks, n_chunks)], src_i)

    pltpu.sync_copy(z_hbm, rows0)
    per_stripe(lambda L: stripe_chunks(L, lambda off, ln: pltpu.sync_copy(
        rows0.at[pl.ds(0, ln)], acc.at[pl.ds(r0 + off, ln)])))
    plsc.subcore_barrier()

    def start(i, dbuf, rbuf, semd, semg):
      pltpu.async_copy(dst_hbm.at[pl.ds(ebase + i * C, C)], dbuf, semd)
      pltpu.async_copy(x_hbm.at[src_i.at[i]], rbuf, semg)

    def finish(dbuf, rbuf, semd, semg):
      pltpu.make_async_copy(dst_hbm.at[pl.ds(0, C)], dbuf, semd).wait()
      pltpu.make_async_copy(x_hbm.at[pl.ds(0, C)], rbuf, semg).wait()
      pltpu.sync_copy(rbuf, acc.at[dbuf], add=True)

    start(0, d0, rows0, semd0, sem0)

    def body2(j, carry):
      i0 = 2 * j
      i1 = i0 + 1
      start(i1, d1, rows1, semd1, sem1)
      finish(d0, rows0, semd0, sem0)

      @pl.when(i1 + 1 < n_chunks)
      def _():
        start(i1 + 1, d0, rows0, semd0, sem0)

      finish(d1, rows1, semd1, sem1)
      return carry

    lax.fori_loop(0, n_chunks // 2, body2, 0)
    plsc.subcore_barrier()

    h0 = cid * NP + r0

    def wb(L):
      def piece(off, ln):
        pltpu.sync_copy(acc.at[pl.ds(r0 + off, ln)], rows0.at[pl.ds(0, ln)])
        pltpu.sync_copy(rows0.at[pl.ds(0, ln)], out_hbm.at[pl.ds(h0 + off, ln)])
      stripe_chunks(L, piece)

    per_stripe(wb)

  return pl.kernel(
      body,
      out_type=[jax.ShapeDtypeStruct((_NC * NP, W), jnp.float32)],
      mesh=mesh,
      scratch_types=[
          pltpu.VMEM_SHARED((NP, W), jnp.float32),
          pltpu.VMEM((n_chunks, C), jnp.int32),
          pltpu.VMEM((C,), jnp.int32),
          pltpu.VMEM((C,), jnp.int32),
          pltpu.VMEM((C, W), jnp.float32),
          pltpu.VMEM((C, W), jnp.float32),
          pltpu.SemaphoreType.DMA,
          pltpu.SemaphoreType.DMA,
          pltpu.SemaphoreType.DMA,
          pltpu.SemaphoreType.DMA,
      ],
  )


def _make_cnt(N, E, W):
  NP = N + _PAD
  n_chunks, EP = _edge_layout(E)
  C = _C
  RS, RS_LAST = _stripes(NP)

  mesh = plsc.VectorSubcoreMesh(core_axis_name="c", subcore_axis_name="s")

  def body(dst_hbm, z_hbm, ones_hbm, out_hbm, acc,
           d0, d1, rows, ones_v, semd0, semd1, sems0, sems1):
    cid = lax.axis_index("c")
    sid = lax.axis_index("s")
    wid = sid * _NC + cid
    r0 = sid * RS
    ebase = wid * n_chunks * C

    def stripe_chunks(L, fn):
      off = 0
      while off < L:
        ln = min(C, L - off)
        fn(off, ln)
        off += ln

    def per_stripe(fn):
      @pl.when(sid != _NS - 1)
      def _():
        fn(RS)

      @pl.when(sid == _NS - 1)
      def _():
        fn(RS_LAST)

    pltpu.sync_copy(z_hbm, rows)
    pltpu.sync_copy(ones_hbm, ones_v)
    per_stripe(lambda L: stripe_chunks(L, lambda off, ln: pltpu.sync_copy(
        rows.at[pl.ds(0, ln)], acc.at[pl.ds(r0 + off, ln)])))
    plsc.subcore_barrier()

    def load_dst(i, dbuf, semd):
      pltpu.async_copy(dst_hbm.at[pl.ds(ebase + i * C, C)], dbuf, semd)

    def wait_dst(dbuf, semd):
      pltpu.make_async_copy(dst_hbm.at[pl.ds(0, C)], dbuf, semd).wait()

    def drain_scatter(dbuf, sems):
      pltpu.make_async_copy(ones_v, acc.at[dbuf], sems).wait()

    load_dst(0, d0, semd0)
    load_dst(1, d1, semd1)

    def body2(j, carry):
      i0 = 2 * j
      i1 = i0 + 1
      wait_dst(d0, semd0)
      pltpu.async_copy(ones_v, acc.at[d0], sems0, add=True)
      wait_dst(d1, semd1)
      pltpu.async_copy(ones_v, acc.at[d1], sems1, add=True)
      drain_scatter(d0, sems0)

      @pl.when(i0 + 2 < n_chunks)
      def _():
        load_dst(i0 + 2, d0, semd0)

      drain_scatter(d1, sems1)

      @pl.when(i1 + 2 < n_chunks)
      def _():
        load_dst(i1 + 2, d1, semd1)

      return carry

    lax.fori_loop(0, n_chunks // 2, body2, 0)
    plsc.subcore_barrier()

    h0 = cid * NP + r0

    def wb(L):
      def piece(off, ln):
        pltpu.sync_copy(acc.at[pl.ds(r0 + off, ln)], rows.at[pl.ds(0, ln)])
        pltpu.sync_copy(rows.at[pl.ds(0, ln)], out_hbm.at[pl.ds(h0 + off, ln)])
      stripe_chunks(L, piece)

    per_stripe(wb)

  return pl.kernel(
      body,
      out_type=[jax.ShapeDtypeStruct((_NC * NP, W), jnp.float32)],
      mesh=mesh,
      scratch_types=[
          pltpu.VMEM_SHARED((NP, W), jnp.float32),
          pltpu.VMEM((C,), jnp.int32),
          pltpu.VMEM((C,), jnp.int32),
          pltpu.VMEM((C, W), jnp.float32),
          pltpu.VMEM((C, W), jnp.float32),
          pltpu.SemaphoreType.DMA,
          pltpu.SemaphoreType.DMA,
          pltpu.SemaphoreType.DMA,
          pltpu.SemaphoreType.DMA,
      ],
  )


def _mm_t(a, w):
  return lax.dot_general(a, w, (((1,), (1,)), ((), ())),
                         preferred_element_type=jnp.float32)


def _tc_layer1(acc2, cnt2, x, Wl, bl, Wr):
  N, D = x.shape
  H = Wl.shape[0]

  def body(acc_ref, cnt_ref, x_ref, wl_ref, bl_ref, wr_ref, o_ref):
    s = acc_ref[0, :N] + acc_ref[1, :N]
    c = cnt_ref[0, :N, 0:1] + cnt_ref[1, :N, 0:1]
    mean = s / jnp.maximum(c, 1.0)
    t = _mm_t(mean, wl_ref[...]) + bl_ref[...] + _mm_t(x_ref[...], wr_ref[...])
    o_ref[...] = jnp.maximum(t, 0.0)

  return pl.pallas_call(
      body,
      out_shape=jax.ShapeDtypeStruct((N, H), jnp.float32),
  )(acc2, cnt2, x, Wl, bl.reshape(1, -1), Wr)


def _tc_layer2(acc2, cnt2, h, Wl, bl, Wr, Wlin, blin):
  N, H = h.shape
  O = Wlin.shape[0]

  def body(acc_ref, cnt_ref, h_ref, wl_ref, bl_ref, wr_ref, wlin_ref,
           blin_ref, h2_ref, xp_ref):
    s = acc_ref[0, :N] + acc_ref[1, :N]
    c = cnt_ref[0, :N, 0:1] + cnt_ref[1, :N, 0:1]
    mean = s / jnp.maximum(c, 1.0)
    h2 = _mm_t(mean, wl_ref[...]) + bl_ref[...] + _mm_t(h_ref[...], wr_ref[...])
    h2_ref[...] = h2
    xp_ref[...] = _mm_t(jnp.maximum(h2, 0.0), wlin_ref[...]) + blin_ref[...]

  return pl.pallas_call(
      body,
      out_shape=[
          jax.ShapeDtypeStruct((N, H), jnp.float32),
          jax.ShapeDtypeStruct((N, O), jnp.float32),
      ],
  )(acc2, cnt2, h, Wl, bl.reshape(1, -1), Wr, Wlin, blin.reshape(1, -1))


def kernel(x, edge_index, Wl1, bl1, Wr1, Wl2, bl2, Wr2, Wlin, blin):
  N, D = x.shape
  H = Wl1.shape[0]
  E = edge_index.shape[1]
  NP = N + _PAD
  n_chunks, EP = _edge_layout(E)

  src = edge_index[0]
  dst = edge_index[1]
  pad = EP - E
  src_p = jnp.concatenate([src, jnp.zeros((pad,), jnp.int32)])
  dst_p = jnp.concatenate(
      [dst, N + (jnp.arange(pad, dtype=jnp.int32) % _PAD)])
  srcr = src_p.reshape(_NW * n_chunks, _C)

  z_d = jnp.zeros((_C, D), jnp.float32)
  z_h = jnp.zeros((_C, H), jnp.float32)
  ones_d = jnp.ones((_C, D), jnp.float32)

  (acc1,) = _make_agg(N, E, D)(x, srcr, dst_p, z_d)
  acc1 = acc1.reshape(_NC, NP, D)
  (cnts,) = _make_cnt(N, E, D)(dst_p, z_d, ones_d)
  cnts = cnts.reshape(_NC, NP, D)
  h = _tc_layer1(acc1, cnts, x, Wl1, bl1, Wr1)
  (acc2,) = _make_agg(N, E, H)(h, srcr, dst_p, z_h)
  acc2 = acc2.reshape(_NC, NP, H)
  h2, x_post = _tc_layer2(acc2, cnts, h, Wl2, bl2, Wr2, Wlin, blin)
  return (h2, x_post)


# speedup vs baseline: 2.9636x; 2.9636x over previous
import jax
import jax.numpy as jnp
from jax import lax
from jax.experimental import pallas as pl
from jax.experimental.pallas import tpu as pltpu
from jax.experimental.pallas import tpu_sc as plsc

_NC = 2
_NS = 16
_NW = _NC * _NS
_C = 128
_PAD = 248


def _edge_layout(E):
  e_per_tile = -(-E // _NW)
  n_chunks = -(-e_per_tile // _C)
  n_chunks = ((n_chunks + 7) // 8) * 8
  if n_chunks % 2:
    n_chunks += 1
  return n_chunks, _NW * n_chunks * _C


def _stripes(NP):
  RS = 8 * ((NP + 8 * _NS - 1) // (8 * _NS))
  RS_LAST = NP - RS * (_NS - 1)
  assert 0 < RS_LAST <= RS
  return RS, RS_LAST


def _make_agg(N, E, W):
  NP = N + _PAD
  n_chunks, EP = _edge_layout(E)
  C = _C
  RS, RS_LAST = _stripes(NP)

  mesh = plsc.VectorSubcoreMesh(core_axis_name="c", subcore_axis_name="s")

  def body(x_hbm, srcr_hbm, dst_hbm, z_hbm, out_hbm, acc,
           src_i, d0, d1, rows0, rows1, sem0, sem1, semd0, semd1):
    cid = lax.axis_index("c")
    sid = lax.axis_index("s")
    wid = sid * _NC + cid
    r0 = sid * RS
    ebase = wid * n_chunks * C

    def stripe_chunks(L, fn):
      off = 0
      while off < L:
        ln = min(C, L - off)
        fn(off, ln)
        off += ln

    def per_stripe(fn):
      @pl.when(sid != _NS - 1)
      def _():
        fn(RS)

      @pl.when(sid == _NS - 1)
      def _():
        fn(RS_LAST)

    pltpu.sync_copy(srcr_hbm.at[pl.ds(wid * n_chunks, n_chunks)], src_i)

    pltpu.sync_copy(z_hbm, rows0)
    per_stripe(lambda L: stripe_chunks(L, lambda off, ln: pltpu.sync_copy(
        rows0.at[pl.ds(0, ln)], acc.at[pl.ds(r0 + off, ln)])))
    plsc.subcore_barrier()

    def start(i, dbuf, rbuf, semd, semg):
      pltpu.async_copy(dst_hbm.at[pl.ds(ebase + i * C, C)], dbuf, semd)
      pltpu.async_copy(x_hbm.at[src_i.at[i]], rbuf, semg)

    def finish(dbuf, rbuf, semd, semg):
      pltpu.make_async_copy(dst_hbm.at[pl.ds(0, C)], dbuf, semd).wait()
      pltpu.make_async_copy(x_hbm.at[pl.ds(0, C)], rbuf, semg).wait()
      pltpu.sync_copy(rbuf, acc.at[dbuf], add=True)

    start(0, d0, rows0, semd0, sem0)

    def body2(j, carry):
      i0 = 2 * j
      i1 = i0 + 1
      start(i1, d1, rows1, semd1, sem1)
      finish(d0, rows0, semd0, sem0)

      @pl.when(i1 + 1 < n_chunks)
      def _():
        start(i1 + 1, d0, rows0, semd0, sem0)

      finish(d1, rows1, semd1, sem1)
      return carry

    lax.fori_loop(0, n_chunks // 2, body2, 0)
    plsc.subcore_barrier()

    h0 = cid * NP + r0

    def wb(L):
      def piece(off, ln):
        pltpu.sync_copy(acc.at[pl.ds(r0 + off, ln)], rows0.at[pl.ds(0, ln)])
        pltpu.sync_copy(rows0.at[pl.ds(0, ln)], out_hbm.at[pl.ds(h0 + off, ln)])
      stripe_chunks(L, piece)

    per_stripe(wb)

  return pl.kernel(
      body,
      out_type=[jax.ShapeDtypeStruct((_NC * NP, W), jnp.float32)],
      mesh=mesh,
      scratch_types=[
          pltpu.VMEM_SHARED((NP, W), jnp.float32),
          pltpu.VMEM((n_chunks, C), jnp.int32),
          pltpu.VMEM((C,), jnp.int32),
          pltpu.VMEM((C,), jnp.int32),
          pltpu.VMEM((C, W), jnp.float32),
          pltpu.VMEM((C, W), jnp.float32),
          pltpu.SemaphoreType.DMA,
          pltpu.SemaphoreType.DMA,
          pltpu.SemaphoreType.DMA,
          pltpu.SemaphoreType.DMA,
      ],
  )


def _make_cnt(N, E, W):
  NP = N + _PAD
  n_chunks, EP = _edge_layout(E)
  C = _C
  RS, RS_LAST = _stripes(NP)

  mesh = plsc.VectorSubcoreMesh(core_axis_name="c", subcore_axis_name="s")

  def body(dst_hbm, z_hbm, ones_hbm, out_hbm, acc,
           d0, d1, rows, ones_v, semd0, semd1, sems0, sems1):
    cid = lax.axis_index("c")
    sid = lax.axis_index("s")
    wid = sid * _NC + cid
    r0 = sid * RS
    ebase = wid * n_chunks * C

    def stripe_chunks(L, fn):
      off = 0
      while off < L:
        ln = min(C, L - off)
        fn(off, ln)
        off += ln

    def per_stripe(fn):
      @pl.when(sid != _NS - 1)
      def _():
        fn(RS)

      @pl.when(sid == _NS - 1)
      def _():
        fn(RS_LAST)

    pltpu.sync_copy(z_hbm, rows)
    pltpu.sync_copy(ones_hbm, ones_v)
    per_stripe(lambda L: stripe_chunks(L, lambda off, ln: pltpu.sync_copy(
        rows.at[pl.ds(0, ln)], acc.at[pl.ds(r0 + off, ln)])))
    plsc.subcore_barrier()

    def load_dst(i, dbuf, semd):
      pltpu.async_copy(dst_hbm.at[pl.ds(ebase + i * C, C)], dbuf, semd)

    def wait_dst(dbuf, semd):
      pltpu.make_async_copy(dst_hbm.at[pl.ds(0, C)], dbuf, semd).wait()

    def drain_scatter(dbuf, sems):
      pltpu.make_async_copy(ones_v, acc.at[dbuf], sems).wait()

    load_dst(0, d0, semd0)
    load_dst(1, d1, semd1)

    def body2(j, carry):
      i0 = 2 * j
      i1 = i0 + 1
      wait_dst(d0, semd0)
      pltpu.async_copy(ones_v, acc.at[d0], sems0, add=True)
      wait_dst(d1, semd1)
      pltpu.async_copy(ones_v, acc.at[d1], sems1, add=True)
      drain_scatter(d0, sems0)

      @pl.when(i0 + 2 < n_chunks)
      def _():
        load_dst(i0 + 2, d0, semd0)

      drain_scatter(d1, sems1)

      @pl.when(i1 + 2 < n_chunks)
      def _():
        load_dst(i1 + 2, d1, semd1)

      return carry

    lax.fori_loop(0, n_chunks // 2, body2, 0)
    plsc.subcore_barrier()

    h0 = cid * NP + r0

    def wb(L):
      def piece(off, ln):
        pltpu.sync_copy(acc.at[pl.ds(r0 + off, ln)], rows.at[pl.ds(0, ln)])
        pltpu.sync_copy(rows.at[pl.ds(0, ln)], out_hbm.at[pl.ds(h0 + off, ln)])
      stripe_chunks(L, piece)

    per_stripe(wb)

  return pl.kernel(
      body,
      out_type=[jax.ShapeDtypeStruct((_NC * NP, W), jnp.float32)],
      mesh=mesh,
      scratch_types=[
          pltpu.VMEM_SHARED((NP, W), jnp.float32),
          pltpu.VMEM((C,), jnp.int32),
          pltpu.VMEM((C,), jnp.int32),
          pltpu.VMEM((C, W), jnp.float32),
          pltpu.VMEM((C, W), jnp.float32),
          pltpu.SemaphoreType.DMA,
          pltpu.SemaphoreType.DMA,
          pltpu.SemaphoreType.DMA,
          pltpu.SemaphoreType.DMA,
      ],
  )


def _mm_t(a, w):
  return lax.dot_general(a, w, (((1,), (1,)), ((), ())),
                         preferred_element_type=jnp.float32)


def _tc_layer1(acc2, cnt2, x, Wl, bl, Wr):
  N, D = x.shape
  H = Wl.shape[0]

  def body(acc_ref, cnt_ref, x_ref, wl_ref, bl_ref, wr_ref, o_ref):
    s = acc_ref[0, :N] + acc_ref[1, :N]
    c = cnt_ref[0, :N, 0:1] + cnt_ref[1, :N, 0:1]
    mean = s / jnp.maximum(c, 1.0)
    t = _mm_t(mean, wl_ref[...]) + bl_ref[...] + _mm_t(x_ref[...], wr_ref[...])
    o_ref[...] = jnp.maximum(t, 0.0)

  return pl.pallas_call(
      body,
      out_shape=jax.ShapeDtypeStruct((N, H), jnp.float32),
  )(acc2, cnt2, x, Wl, bl.reshape(1, -1), Wr)


def _tc_layer2(acc2, cnt2, h, Wl, bl, Wr, Wlin, blin):
  N, H = h.shape
  O = Wlin.shape[0]

  def body(acc_ref, cnt_ref, h_ref, wl_ref, bl_ref, wr_ref, wlin_ref,
           blin_ref, h2_ref, xp_ref):
    s = acc_ref[0, :N] + acc_ref[1, :N]
    c = cnt_ref[0, :N, 0:1] + cnt_ref[1, :N, 0:1]
    mean = s / jnp.maximum(c, 1.0)
    h2 = _mm_t(mean, wl_ref[...]) + bl_ref[...] + _mm_t(h_ref[...], wr_ref[...])
    h2_ref[...] = h2
    xp_ref[...] = _mm_t(jnp.maximum(h2, 0.0), wlin_ref[...]) + blin_ref[...]

  return pl.pallas_call(
      body,
      out_shape=[
          jax.ShapeDtypeStruct((N, H), jnp.float32),
          jax.ShapeDtypeStruct((N, O), jnp.float32),
      ],
  )(acc2, cnt2, h, Wl, bl.reshape(1, -1), Wr, Wlin, blin.reshape(1, -1))


def kernel(x, edge_index, Wl1, bl1, Wr1, Wl2, bl2, Wr2, Wlin, blin):
  N, D = x.shape
  H = Wl1.shape[0]
  E = edge_index.shape[1]
  NP = N + _PAD
  n_chunks, EP = _edge_layout(E)

  src = edge_index[0]
  dst = edge_index[1]
  e_real = -(-E // _NW)
  fpad = _NW * e_real - E
  if fpad:
    src = jnp.concatenate([src, jnp.zeros((fpad,), jnp.int32)])
    dst = jnp.concatenate(
        [dst, N + (jnp.arange(fpad, dtype=jnp.int32) % _PAD)])
  ppt = n_chunks * _C - e_real
  iota_p = jnp.arange(ppt, dtype=jnp.int32)[None, :]
  tile_i = jnp.arange(_NW, dtype=jnp.int32)[:, None]
  src_pad = (iota_p + tile_i * ppt) % N
  dst_pad = N + (iota_p + tile_i * 7) % _PAD
  src_p = jnp.concatenate([src.reshape(_NW, e_real), src_pad], axis=1)
  dst_p = jnp.concatenate([dst.reshape(_NW, e_real), dst_pad],
                          axis=1).reshape(-1)
  srcr = src_p.reshape(_NW * n_chunks, _C)

  z_d = jnp.zeros((_C, D), jnp.float32)
  z_h = jnp.zeros((_C, H), jnp.float32)
  ones_d = jnp.ones((_C, D), jnp.float32)

  (acc1,) = _make_agg(N, E, D)(x, srcr, dst_p, z_d)
  acc1 = acc1.reshape(_NC, NP, D)
  (cnts,) = _make_cnt(N, E, D)(dst_p, z_d, ones_d)
  cnts = cnts.reshape(_NC, NP, D)
  h = _tc_layer1(acc1, cnts, x, Wl1, bl1, Wr1)
  (acc2,) = _make_agg(N, E, H)(h, srcr, dst_p, z_h)
  acc2 = acc2.reshape(_NC, NP, H)
  h2, x_post = _tc_layer2(acc2, cnts, h, Wl2, bl2, Wr2, Wlin, blin)
  return (h2, x_post)
